# baseline (speedup 1.0000x reference)
import functools

import jax
import jax.numpy as jnp
from jax import lax
from jax.experimental import pallas as pl
from jax.experimental.pallas import tpu as pltpu
from jax.experimental.pallas import tpu_sc as plsc

N = 10000
E = 320000
D = 128
CHUNK = 64
NCHUNKS = E // CHUNK
NC, NS = 2, 16
NW = NC * NS
KMAX = (NCHUNKS + NW - 1) // NW
CH1 = 128
NCH1 = E // CH1
SLOTS1 = 81
NPAD = 10240
ROWS_PER_TILE = NPAD // NS
XW = 16
AW = D + XW


def _ln_rows(xb, g, b, eps=1e-5):
    mu = jnp.mean(xb, axis=-1, keepdims=True)
    var = jnp.mean((xb - mu) ** 2, axis=-1, keepdims=True)
    return (xb - mu) / jnp.sqrt(var + eps) * g + b


def _prep_node_body(h_ref, w1a_ref, w1b_ref, a_ref, b_ref):
    h = h_ref[...]
    a_ref[...] = jnp.dot(h, w1a_ref[...], preferred_element_type=jnp.float32)
    b_ref[...] = jnp.dot(h, w1b_ref[...], preferred_element_type=jnp.float32)


def _fold_w_body(w2_ref, b2_ref, pxw1_ref, pxb1_ref, wx_ref, bx_ref):
    wx_ref[...] = jnp.dot(w2_ref[...], pxw1_ref[...],
                          preferred_element_type=jnp.float32)
    bx_ref[...] = jnp.dot(b2_ref[...], pxw1_ref[...],
                          preferred_element_type=jnp.float32) + pxb1_ref[...]


def _make_sc1_body(nch1, nbody):
  def _sc1_body(a_h, b_h, xp_h, u_h, v_h, pre_h, df_h, *scr):
    sets = (scr[0:7], scr[7:14], scr[14:21])
    sg = scr[21:24]
    sw = scr[24:27]
    wid = lax.axis_index("s") * NC + lax.axis_index("c")

    def fire_gathers(k, b):
        iu, iv, au, bv, xu, xv, df = sets[b]
        chunk = k * NW + wid

        @pl.when(chunk < nch1)
        def _():
            pltpu.sync_copy(u_h.at[chunk], iu)
            pltpu.sync_copy(v_h.at[chunk], iv)
            pltpu.async_copy(a_h.at[iu], au, sg[b])
            pltpu.async_copy(b_h.at[iv], bv, sg[b])
            pltpu.async_copy(xp_h.at[iu], xu, sg[b])
            pltpu.async_copy(xp_h.at[iv], xv, sg[b])

    def drain_writes(k, b):
        iu, iv, au, bv, xu, xv, df = sets[b]
        chunk = jnp.maximum(k, 0) * NW + wid

        @pl.when((k >= 0) & (chunk < nch1))
        def _():
            row = pl.ds(chunk * CH1, CH1)
            pltpu.make_async_copy(au, pre_h.at[row], sw[b]).wait()
            pltpu.make_async_copy(df, df_h.at[row], sw[b]).wait()

    def process(k, b):
        iu, iv, au, bv, xu, xv, df = sets[b]
        chunk = k * NW + wid

        @pl.when(chunk < nch1)
        def _():
            pltpu.make_async_copy(a_h.at[iu], au, sg[b]).wait()
            pltpu.make_async_copy(b_h.at[iv], bv, sg[b]).wait()
            pltpu.make_async_copy(xp_h.at[iu], xu, sg[b]).wait()
            pltpu.make_async_copy(xp_h.at[iv], xv, sg[b]).wait()

            def _row(r, _):
                for j in range(D // 16):
                    sl = pl.ds(j * 16, 16)
                    au[r, sl] = au[r, sl] + bv[r, sl]
                df[r, :] = xu[r, :] - xv[r, :]
                return 0
            lax.fori_loop(0, CH1, _row, 0)

            row = pl.ds(chunk * CH1, CH1)
            pltpu.async_copy(au, pre_h.at[row], sw[b])
            pltpu.async_copy(df, df_h.at[row], sw[b])

    fire_gathers(0, 0)
    fire_gathers(1, 1)

    def _body(t, _):
        for b in range(3):
            k = t * 3 + b
            process(k, b)
            drain_writes(k - 1, (b + 2) % 3)
            fire_gathers(k + 2, (b + 2) % 3)
        return 0
    lax.fori_loop(0, nbody, _body, 0)

  return _sc1_body


def _edge_x_body(pre_ref, ea_ref, df_ref, w1c_ref, b1_ref, wx_ref, bx_ref,
                 w2r_ref, b2_ref, m1_ref, stp_ref):
    pre = (pre_ref[...]
           + jnp.dot(ea_ref[...], w1c_ref[...],
                     preferred_element_type=jnp.float32)
           + b1_ref[...])
    m1 = pre * jax.nn.sigmoid(pre)
    m1_ref[...] = m1
    t = jnp.dot(m1, wx_ref[...], preferred_element_type=jnp.float32) + bx_ref[...]
    t = t * jax.nn.sigmoid(t)
    w = jnp.tanh(jnp.sum(t * w2r_ref[...], axis=-1, keepdims=True)
                 + b2_ref[...]) * 0.1
    df = df_ref[...]
    dist = jnp.maximum(jnp.sqrt(jnp.sum(df * df, axis=-1, keepdims=True)), 1e-8)
    lane = lax.broadcasted_iota(jnp.int32, df.shape, 1)
    ind = jnp.where((lane == 3) | (lane == 4), 1.0, 0.0)
    stp_ref[...] = w * df / dist + ind


def _make_sc2_body(nchunks, nbody2):
  def _sc2_body(m1_h, st_h, u_h, v_h, ms_h, xa_h, *scr):
    sets = (scr[0:5], scr[5:10])
    sl_sem = scr[10:12]
    ms_s, xa_s = scr[12], scr[13]
    cid = lax.axis_index("c")
    sid = lax.axis_index("s")
    wid = sid * NC + cid

    zero16 = jnp.zeros((16,), jnp.float32)
    m1c0, st0 = sets[0][2], sets[0][3]

    def _zrow(r, _):
        for j in range(D // 16):
            m1c0[r, pl.ds(j * 16, 16)] = zero16
        st0[r, :] = zero16
        return 0
    lax.fori_loop(0, CHUNK, _zrow, 0)
    for i in range(ROWS_PER_TILE // CHUNK):
        dst = pl.ds(sid * ROWS_PER_TILE + i * CHUNK, CHUNK)
        pltpu.sync_copy(m1c0.at[pl.ds(0, CHUNK), pl.ds(0, D)], ms_s.at[dst])
        pltpu.sync_copy(st0.at[pl.ds(0, CHUNK), pl.ds(0, XW)], xa_s.at[dst])

    plsc.subcore_barrier()

    lane = lax.broadcasted_iota(jnp.int32, (16,), 0)
    negmask = jnp.where(lane < 3, -1.0,
                        jnp.where(lane == 3, 1.0, 0.0)).astype(jnp.float32)

    def fire_loads(k, b):
        iu, iv, m1c, st16, stn16 = sets[b]
        chunk = k * NW + wid

        @pl.when(chunk < nchunks)
        def _():
            pltpu.sync_copy(u_h.at[chunk], iu)
            pltpu.sync_copy(v_h.at[chunk], iv)
            row = pl.ds(chunk * CHUNK, CHUNK)
            pltpu.async_copy(m1_h.at[row], m1c, sl_sem[b])
            pltpu.async_copy(st_h.at[row], st16, sl_sem[b])

    def work(k, b):
        iu, iv, m1c, st16, stn16 = sets[b]
        chunk = k * NW + wid

        @pl.when(chunk < nchunks)
        def _():
            row = pl.ds(chunk * CHUNK, CHUNK)
            pltpu.make_async_copy(m1_h.at[row], m1c, sl_sem[b]).wait()
            pltpu.make_async_copy(st_h.at[row], st16, sl_sem[b]).wait()

            def _row(r, _):
                stn16[r, :] = st16[r, :] * negmask
                return 0
            lax.fori_loop(0, CHUNK, _row, 0)

            pltpu.sync_copy(m1c, ms_s.at[iu], add=True)
            pltpu.sync_copy(st16, xa_s.at[iu], add=True)
            pltpu.sync_copy(stn16, xa_s.at[iv], add=True)

    fire_loads(0, 0)

    def _body(t, _):
        for b in range(2):
            k = t * 2 + b
            fire_loads(k + 1, 1 - b)
            work(k, b)
        return 0
    lax.fori_loop(0, nbody2, _body, 0)

    plsc.subcore_barrier()
    src = pl.ds(sid * ROWS_PER_TILE, ROWS_PER_TILE)
    pltpu.sync_copy(ms_s.at[src], ms_h.at[cid, src])
    pltpu.sync_copy(xa_s.at[src], xa_h.at[cid, src])

  return _sc2_body


def _make_node_body(seg):
  def _node_body(h_ref, xp_ref, *rest):
    ms_refs = rest[0:seg]
    xa_refs = rest[seg:2 * seg]
    (w2_ref, b2_ref, wha_ref, whb_ref, bh1_ref, wh2_ref, bh2_ref,
     lnhg_ref, lnhb_ref, lnmg_ref, lnmb_ref, hn_ref, xn_ref) = rest[2 * seg:]
    ms = ms_refs[0][0] + ms_refs[0][1]
    xa = xa_refs[0][0] + xa_refs[0][1]
    for i in range(1, seg):
        ms = ms + ms_refs[i][0] + ms_refs[i][1]
        xa = xa + xa_refs[i][0] + xa_refs[i][1]
    deg = xa[:, 4:5]
    m_sum = (jnp.dot(ms, w2_ref[...], preferred_element_type=jnp.float32)
             + deg * b2_ref[...])
    m_i = m_sum / jnp.maximum(deg, 1.0)
    m_i = _ln_rows(m_i, lnmg_ref[...], lnmb_ref[...])
    h = h_ref[...]
    h_in = _ln_rows(h, lnhg_ref[...], lnhb_ref[...])
    pre = (jnp.dot(h_in, wha_ref[...], preferred_element_type=jnp.float32)
           + jnp.dot(m_i, whb_ref[...], preferred_element_type=jnp.float32)
           + bh1_ref[...])
    upd = (jnp.dot(pre * jax.nn.sigmoid(pre), wh2_ref[...],
                   preferred_element_type=jnp.float32) + bh2_ref[...])
    hn_ref[...] = h + 0.3 * upd

    degx = jnp.maximum(xa[:, 3:4], 1.0)
    xn_ref[...] = xp_ref[...] + xa / degx

  return _node_body


def kernel(x, h, edge_index, edge_attr, pm_w1, pm_b1, pm_w2, pm_b2,
           ph_w1, ph_b1, ph_w2, ph_b2, px_w1, px_b1, px_w2, px_b2,
           lnh_g, lnh_b, lnm_g, lnm_b):
    f32 = jnp.float32
    xp = jnp.pad(x, ((0, 0), (0, XW - 3)))

    w1a = pm_w1[:D]
    w1b = pm_w1[D:2 * D]
    w1c = pm_w1[2 * D:]
    b1r = pm_b1.reshape(1, D)

    nb = 10
    bn = N // nb
    a_nd, b_nd = pl.pallas_call(
        _prep_node_body,
        grid=(nb,),
        in_specs=[
            pl.BlockSpec((bn, D), lambda i: (i, 0)),
            pl.BlockSpec((D, D), lambda i: (0, 0)),
            pl.BlockSpec((D, D), lambda i: (0, 0)),
        ],
        out_specs=[
            pl.BlockSpec((bn, D), lambda i: (i, 0)),
            pl.BlockSpec((bn, D), lambda i: (i, 0)),
        ],
        out_shape=[
            jax.ShapeDtypeStruct((N, D), f32),
            jax.ShapeDtypeStruct((N, D), f32),
        ],
    )(h, w1a, w1b)

    eb = 200
    be = E // eb

    wx, bx = pl.pallas_call(
        _fold_w_body,
        out_shape=[
            jax.ShapeDtypeStruct((D, D), f32),
            jax.ShapeDtypeStruct((1, D), f32),
        ],
    )(pm_w2, pm_b2.reshape(1, D), px_w1, px_b1.reshape(1, D))

    mesh = plsc.VectorSubcoreMesh(core_axis_name="c", subcore_axis_name="s",
                                  num_cores=NC, num_subcores=NS)
    sc_params = pltpu.CompilerParams(use_tc_tiling_on_sc=False)
    SEG = 2
    E2 = E // SEG
    nch1_h = E2 // CH1
    kmax1_h = (nch1_h + NW - 1) // NW
    nbody1_h = (kmax1_h + 2 + 2) // 3
    nch2_h = E2 // CHUNK
    kmax2_h = (nch2_h + NW - 1) // NW
    u_all = edge_index[0].astype(jnp.int32)
    v_all = edge_index[1].astype(jnp.int32)
    set_types = [
        pltpu.VMEM((CH1,), jnp.int32),
        pltpu.VMEM((CH1,), jnp.int32),
        pltpu.VMEM((CH1, D), f32),
        pltpu.VMEM((CH1, D), f32),
        pltpu.VMEM((CH1, XW), f32),
        pltpu.VMEM((CH1, XW), f32),
        pltpu.VMEM((CH1, XW), f32),
    ]
    sc1_body = _make_sc1_body(nch1_h, nbody1_h)
    sc2_body = _make_sc2_body(nch2_h, (kmax2_h + 1) // 2)
    set2_types = [
        pltpu.VMEM((CHUNK,), jnp.int32),
        pltpu.VMEM((CHUNK,), jnp.int32),
        pltpu.VMEM((CHUNK, D), f32),
        pltpu.VMEM((CHUNK, XW), f32),
        pltpu.VMEM((CHUNK, XW), f32),
    ]
    eb = 200 // SEG
    be = E2 // eb

    msum_ps, xacc_ps = [], []
    for hf in range(SEG):
        sl = slice(hf * E2, (hf + 1) * E2)
        u1 = u_all[sl].reshape(nch1_h, CH1)
        v1 = v_all[sl].reshape(nch1_h, CH1)
        u2 = u_all[sl].reshape(nch2_h, CHUNK)
        v2 = v_all[sl].reshape(nch2_h, CHUNK)

        pre, diff = pl.kernel(
            sc1_body,
            compiler_params=sc_params,
            out_type=[
                jax.ShapeDtypeStruct((E2, D), f32),
                jax.ShapeDtypeStruct((E2, XW), f32),
            ],
            mesh=mesh,
            scratch_types=(set_types * 3
                           + [pltpu.SemaphoreType.DMA] * 6),
        )(a_nd, b_nd, xp, u1, v1)

        m1, step = pl.pallas_call(
            _edge_x_body,
            grid=(eb,),
            in_specs=[
                pl.BlockSpec((be, D), lambda i: (i, 0)),
                pl.BlockSpec((be, 16), lambda i: (i, 0)),
                pl.BlockSpec((be, XW), lambda i: (i, 0)),
                pl.BlockSpec((16, D), lambda i: (0, 0)),
                pl.BlockSpec((1, D), lambda i: (0, 0)),
                pl.BlockSpec((D, D), lambda i: (0, 0)),
                pl.BlockSpec((1, D), lambda i: (0, 0)),
                pl.BlockSpec((1, D), lambda i: (0, 0)),
                pl.BlockSpec((1, 1), lambda i: (0, 0)),
            ],
            out_specs=[
                pl.BlockSpec((be, D), lambda i: (i, 0)),
                pl.BlockSpec((be, XW), lambda i: (i, 0)),
            ],
            out_shape=[
                jax.ShapeDtypeStruct((E2, D), f32),
                jax.ShapeDtypeStruct((E2, XW), f32),
            ],
        )(pre, edge_attr[sl], diff, w1c, b1r, wx, bx,
          px_w2.reshape(1, D), px_b2.reshape(1, 1))

        msum_p, xacc_p = pl.kernel(
            sc2_body,
            compiler_params=sc_params,
            out_type=[
                jax.ShapeDtypeStruct((NC, NPAD, D), f32),
                jax.ShapeDtypeStruct((NC, NPAD, XW), f32),
            ],
            mesh=mesh,
            scratch_types=(set2_types * 2
                           + [pltpu.SemaphoreType.DMA] * 2
                           + [pltpu.VMEM_SHARED((NPAD, D), f32),
                              pltpu.VMEM_SHARED((NPAD, XW), f32)]),
        )(m1, step, u2, v2)
        msum_ps.append(msum_p)
        xacc_ps.append(xacc_p)

    h_new, xn16 = pl.pallas_call(
        _make_node_body(SEG),
        grid=(nb,),
        in_specs=(
            [pl.BlockSpec((bn, D), lambda i: (i, 0)),
             pl.BlockSpec((bn, XW), lambda i: (i, 0))]
            + [pl.BlockSpec((NC, bn, D), lambda i: (0, i, 0))] * SEG
            + [pl.BlockSpec((NC, bn, XW), lambda i: (0, i, 0))] * SEG
            + [pl.BlockSpec((D, D), lambda i: (0, 0)),
               pl.BlockSpec((1, D), lambda i: (0, 0)),
               pl.BlockSpec((D, D), lambda i: (0, 0)),
               pl.BlockSpec((D, D), lambda i: (0, 0)),
               pl.BlockSpec((1, D), lambda i: (0, 0)),
               pl.BlockSpec((D, D), lambda i: (0, 0)),
               pl.BlockSpec((1, D), lambda i: (0, 0)),
               pl.BlockSpec((1, D), lambda i: (0, 0)),
               pl.BlockSpec((1, D), lambda i: (0, 0)),
               pl.BlockSpec((1, D), lambda i: (0, 0)),
               pl.BlockSpec((1, D), lambda i: (0, 0))]
        ),
        out_specs=[
            pl.BlockSpec((bn, D), lambda i: (i, 0)),
            pl.BlockSpec((bn, XW), lambda i: (i, 0)),
        ],
        out_shape=[
            jax.ShapeDtypeStruct((N, D), f32),
            jax.ShapeDtypeStruct((N, XW), f32),
        ],
    )(h, xp, *msum_ps, *xacc_ps,
      pm_w2, pm_b2.reshape(1, D),
      ph_w1[:D], ph_w1[D:], ph_b1.reshape(1, D), ph_w2, ph_b2.reshape(1, D),
      lnh_g.reshape(1, D), lnh_b.reshape(1, D),
      lnm_g.reshape(1, D), lnm_b.reshape(1, D))

    return xn16[:, :3], h_new


# speedup vs baseline: 1.0877x; 1.0877x over previous
import functools

import jax
import jax.numpy as jnp
from jax import lax
from jax.experimental import pallas as pl
from jax.experimental.pallas import tpu as pltpu
from jax.experimental.pallas import tpu_sc as plsc

N = 10000
E = 320000
D = 128
CHUNK = 64
NCHUNKS = E // CHUNK
NC, NS = 2, 16
NW = NC * NS
KMAX = (NCHUNKS + NW - 1) // NW
CH1 = 128
NCH1 = E // CH1
SLOTS1 = 81
NPAD = 10240
ROWS_PER_TILE = NPAD // NS
XW = 16
AW = D + XW


def _ln_rows(xb, g, b, eps=1e-5):
    mu = jnp.mean(xb, axis=-1, keepdims=True)
    var = jnp.mean((xb - mu) ** 2, axis=-1, keepdims=True)
    return (xb - mu) / jnp.sqrt(var + eps) * g + b


def _prep_node_body(h_ref, w1a_ref, w1b_ref, a_ref, b_ref):
    h = h_ref[...]
    a_ref[...] = jnp.dot(h, w1a_ref[...], preferred_element_type=jnp.float32)
    b_ref[...] = jnp.dot(h, w1b_ref[...], preferred_element_type=jnp.float32)


def _fold_w_body(w2_ref, b2_ref, pxw1_ref, pxb1_ref, wx_ref, bx_ref):
    wx_ref[...] = jnp.dot(w2_ref[...], pxw1_ref[...],
                          preferred_element_type=jnp.float32)
    bx_ref[...] = jnp.dot(b2_ref[...], pxw1_ref[...],
                          preferred_element_type=jnp.float32) + pxb1_ref[...]


def _make_sc1_body(nch1, nbody):
  def _sc1_body(a_h, b_h, xp_h, u_h, v_h, pre_h, df_h, *scr):
    sets = (scr[0:7], scr[7:14], scr[14:21])
    sg = scr[21:24]
    sw = scr[24:27]
    wid = lax.axis_index("s") * NC + lax.axis_index("c")

    def fire_gathers(k, b):
        iu, iv, au, bv, xu, xv, df = sets[b]
        chunk = k * NW + wid

        @pl.when(chunk < nch1)
        def _():
            pltpu.sync_copy(u_h.at[chunk], iu)
            pltpu.sync_copy(v_h.at[chunk], iv)
            pltpu.async_copy(a_h.at[iu], au, sg[b])
            pltpu.async_copy(b_h.at[iv], bv, sg[b])
            pltpu.async_copy(xp_h.at[iu], xu, sg[b])
            pltpu.async_copy(xp_h.at[iv], xv, sg[b])

    def drain_writes(k, b):
        iu, iv, au, bv, xu, xv, df = sets[b]
        chunk = jnp.maximum(k, 0) * NW + wid

        @pl.when((k >= 0) & (chunk < nch1))
        def _():
            row = pl.ds(chunk * CH1, CH1)
            pltpu.make_async_copy(au, pre_h.at[row], sw[b]).wait()
            pltpu.make_async_copy(df, df_h.at[row], sw[b]).wait()

    def process(k, b):
        iu, iv, au, bv, xu, xv, df = sets[b]
        chunk = k * NW + wid

        @pl.when(chunk < nch1)
        def _():
            pltpu.make_async_copy(a_h.at[iu], au, sg[b]).wait()
            pltpu.make_async_copy(b_h.at[iv], bv, sg[b]).wait()
            pltpu.make_async_copy(xp_h.at[iu], xu, sg[b]).wait()
            pltpu.make_async_copy(xp_h.at[iv], xv, sg[b]).wait()

            def _row(r, _):
                for j in range(D // 16):
                    sl = pl.ds(j * 16, 16)
                    au[r, sl] = au[r, sl] + bv[r, sl]
                df[r, :] = xu[r, :] - xv[r, :]
                return 0
            lax.fori_loop(0, CH1, _row, 0)

            row = pl.ds(chunk * CH1, CH1)
            pltpu.async_copy(au, pre_h.at[row], sw[b])
            pltpu.async_copy(df, df_h.at[row], sw[b])

    fire_gathers(0, 0)
    fire_gathers(1, 1)

    def _body(t, _):
        for b in range(3):
            k = t * 3 + b
            process(k, b)
            drain_writes(k - 1, (b + 2) % 3)
            fire_gathers(k + 2, (b + 2) % 3)
        return 0
    lax.fori_loop(0, nbody, _body, 0)

  return _sc1_body


def _edge_x_body(pre_ref, ea_ref, df_ref, w1c_ref, b1_ref, wx_ref, bx_ref,
                 w2r_ref, b2_ref, m1_ref, stp_ref):
    pre = (pre_ref[...]
           + jnp.dot(ea_ref[...], w1c_ref[...],
                     preferred_element_type=jnp.float32)
           + b1_ref[...])
    m1 = pre * jax.nn.sigmoid(pre)
    m1_ref[...] = m1
    t = jnp.dot(m1, wx_ref[...], preferred_element_type=jnp.float32) + bx_ref[...]
    t = t * jax.nn.sigmoid(t)
    w = jnp.tanh(jnp.sum(t * w2r_ref[...], axis=-1, keepdims=True)
                 + b2_ref[...]) * 0.1
    df = df_ref[...]
    dist = jnp.maximum(jnp.sqrt(jnp.sum(df * df, axis=-1, keepdims=True)), 1e-8)
    lane = lax.broadcasted_iota(jnp.int32, df.shape, 1)
    ind = jnp.where((lane == 3) | (lane == 4), 1.0, 0.0)
    stp_ref[...] = w * df / dist + ind


def _make_sc2_body(nchunks, nbody2):
  def _sc2_body(m1_h, st_h, u_h, v_h, ms_h, xa_h, *scr):
    sets = (scr[0:5], scr[5:10])
    sl_sem = scr[10:12]
    ms_s, xa_s = scr[12], scr[13]
    cid = lax.axis_index("c")
    sid = lax.axis_index("s")
    wid = sid * NC + cid

    zero16 = jnp.zeros((16,), jnp.float32)
    m1c0, st0 = sets[0][2], sets[0][3]

    def _zrow(r, _):
        for j in range(D // 16):
            m1c0[r, pl.ds(j * 16, 16)] = zero16
        st0[r, :] = zero16
        return 0
    lax.fori_loop(0, CHUNK, _zrow, 0)
    for i in range(ROWS_PER_TILE // CHUNK):
        dst = pl.ds(sid * ROWS_PER_TILE + i * CHUNK, CHUNK)
        pltpu.sync_copy(m1c0.at[pl.ds(0, CHUNK), pl.ds(0, D)], ms_s.at[dst])
        pltpu.sync_copy(st0.at[pl.ds(0, CHUNK), pl.ds(0, XW)], xa_s.at[dst])

    plsc.subcore_barrier()

    lane = lax.broadcasted_iota(jnp.int32, (16,), 0)
    negmask = jnp.where(lane < 3, -1.0,
                        jnp.where(lane == 3, 1.0, 0.0)).astype(jnp.float32)

    def fire_loads(k, b):
        iu, iv, m1c, st16, stn16 = sets[b]
        chunk = k * NW + wid

        @pl.when(chunk < nchunks)
        def _():
            pltpu.sync_copy(u_h.at[chunk], iu)
            pltpu.sync_copy(v_h.at[chunk], iv)
            row = pl.ds(chunk * CHUNK, CHUNK)
            pltpu.async_copy(m1_h.at[row], m1c, sl_sem[b])
            pltpu.async_copy(st_h.at[row], st16, sl_sem[b])

    def work(k, b):
        iu, iv, m1c, st16, stn16 = sets[b]
        chunk = k * NW + wid

        @pl.when(chunk < nchunks)
        def _():
            row = pl.ds(chunk * CHUNK, CHUNK)
            pltpu.make_async_copy(m1_h.at[row], m1c, sl_sem[b]).wait()
            pltpu.make_async_copy(st_h.at[row], st16, sl_sem[b]).wait()

            def _row(r, _):
                stn16[r, :] = st16[r, :] * negmask
                return 0
            lax.fori_loop(0, CHUNK, _row, 0)

            pltpu.sync_copy(m1c, ms_s.at[iu], add=True)
            pltpu.sync_copy(st16, xa_s.at[iu], add=True)
            pltpu.sync_copy(stn16, xa_s.at[iv], add=True)

    fire_loads(0, 0)

    def _body(t, _):
        for b in range(2):
            k = t * 2 + b
            fire_loads(k + 1, 1 - b)
            work(k, b)
        return 0
    lax.fori_loop(0, nbody2, _body, 0)

    plsc.subcore_barrier()
    src = pl.ds(sid * ROWS_PER_TILE, ROWS_PER_TILE)
    pltpu.sync_copy(ms_s.at[src], ms_h.at[cid, src])
    pltpu.sync_copy(xa_s.at[src], xa_h.at[cid, src])

  return _sc2_body


def _make_node_body(seg):
  def _node_body(h_ref, xp_ref, *rest):
    ms_refs = rest[0:seg]
    xa_refs = rest[seg:2 * seg]
    (w2_ref, b2_ref, wha_ref, whb_ref, bh1_ref, wh2_ref, bh2_ref,
     lnhg_ref, lnhb_ref, lnmg_ref, lnmb_ref, hn_ref, xn_ref) = rest[2 * seg:]
    ms = ms_refs[0][0] + ms_refs[0][1]
    xa = xa_refs[0][0] + xa_refs[0][1]
    for i in range(1, seg):
        ms = ms + ms_refs[i][0] + ms_refs[i][1]
        xa = xa + xa_refs[i][0] + xa_refs[i][1]
    deg = xa[:, 4:5]
    m_sum = (jnp.dot(ms, w2_ref[...], preferred_element_type=jnp.float32)
             + deg * b2_ref[...])
    m_i = m_sum / jnp.maximum(deg, 1.0)
    m_i = _ln_rows(m_i, lnmg_ref[...], lnmb_ref[...])
    h = h_ref[...]
    h_in = _ln_rows(h, lnhg_ref[...], lnhb_ref[...])
    pre = (jnp.dot(h_in, wha_ref[...], preferred_element_type=jnp.float32)
           + jnp.dot(m_i, whb_ref[...], preferred_element_type=jnp.float32)
           + bh1_ref[...])
    upd = (jnp.dot(pre * jax.nn.sigmoid(pre), wh2_ref[...],
                   preferred_element_type=jnp.float32) + bh2_ref[...])
    hn_ref[...] = h + 0.3 * upd

    degx = jnp.maximum(xa[:, 3:4], 1.0)
    xn_ref[...] = xp_ref[...] + xa / degx

  return _node_body


def kernel(x, h, edge_index, edge_attr, pm_w1, pm_b1, pm_w2, pm_b2,
           ph_w1, ph_b1, ph_w2, ph_b2, px_w1, px_b1, px_w2, px_b2,
           lnh_g, lnh_b, lnm_g, lnm_b):
    f32 = jnp.float32
    xp = jnp.pad(x, ((0, 0), (0, XW - 3)))

    w1a = pm_w1[:D]
    w1b = pm_w1[D:2 * D]
    w1c = pm_w1[2 * D:]
    b1r = pm_b1.reshape(1, D)

    nb = 10
    bn = N // nb
    a_nd, b_nd = pl.pallas_call(
        _prep_node_body,
        grid=(nb,),
        in_specs=[
            pl.BlockSpec((bn, D), lambda i: (i, 0)),
            pl.BlockSpec((D, D), lambda i: (0, 0)),
            pl.BlockSpec((D, D), lambda i: (0, 0)),
        ],
        out_specs=[
            pl.BlockSpec((bn, D), lambda i: (i, 0)),
            pl.BlockSpec((bn, D), lambda i: (i, 0)),
        ],
        out_shape=[
            jax.ShapeDtypeStruct((N, D), f32),
            jax.ShapeDtypeStruct((N, D), f32),
        ],
    )(h, w1a, w1b)

    eb = 200
    be = E // eb

    wx, bx = pl.pallas_call(
        _fold_w_body,
        out_shape=[
            jax.ShapeDtypeStruct((D, D), f32),
            jax.ShapeDtypeStruct((1, D), f32),
        ],
    )(pm_w2, pm_b2.reshape(1, D), px_w1, px_b1.reshape(1, D))

    mesh = plsc.VectorSubcoreMesh(core_axis_name="c", subcore_axis_name="s",
                                  num_cores=NC, num_subcores=NS)
    sc_params = pltpu.CompilerParams(use_tc_tiling_on_sc=False)
    SEG = 5
    E2 = E // SEG
    nch1_h = E2 // CH1
    kmax1_h = (nch1_h + NW - 1) // NW
    nbody1_h = (kmax1_h + 2 + 2) // 3
    nch2_h = E2 // CHUNK
    kmax2_h = (nch2_h + NW - 1) // NW
    u_all = edge_index[0].astype(jnp.int32)
    v_all = edge_index[1].astype(jnp.int32)
    set_types = [
        pltpu.VMEM((CH1,), jnp.int32),
        pltpu.VMEM((CH1,), jnp.int32),
        pltpu.VMEM((CH1, D), f32),
        pltpu.VMEM((CH1, D), f32),
        pltpu.VMEM((CH1, XW), f32),
        pltpu.VMEM((CH1, XW), f32),
        pltpu.VMEM((CH1, XW), f32),
    ]
    sc1_body = _make_sc1_body(nch1_h, nbody1_h)
    sc2_body = _make_sc2_body(nch2_h, (kmax2_h + 1) // 2)
    set2_types = [
        pltpu.VMEM((CHUNK,), jnp.int32),
        pltpu.VMEM((CHUNK,), jnp.int32),
        pltpu.VMEM((CHUNK, D), f32),
        pltpu.VMEM((CHUNK, XW), f32),
        pltpu.VMEM((CHUNK, XW), f32),
    ]
    eb = 200 // SEG
    be = E2 // eb

    msum_ps, xacc_ps = [], []
    for hf in range(SEG):
        sl = slice(hf * E2, (hf + 1) * E2)
        u1 = u_all[sl].reshape(nch1_h, CH1)
        v1 = v_all[sl].reshape(nch1_h, CH1)
        u2 = u_all[sl].reshape(nch2_h, CHUNK)
        v2 = v_all[sl].reshape(nch2_h, CHUNK)

        pre, diff = pl.kernel(
            sc1_body,
            compiler_params=sc_params,
            out_type=[
                jax.ShapeDtypeStruct((E2, D), f32),
                jax.ShapeDtypeStruct((E2, XW), f32),
            ],
            mesh=mesh,
            scratch_types=(set_types * 3
                           + [pltpu.SemaphoreType.DMA] * 6),
        )(a_nd, b_nd, xp, u1, v1)

        m1, step = pl.pallas_call(
            _edge_x_body,
            grid=(eb,),
            in_specs=[
                pl.BlockSpec((be, D), lambda i: (i, 0)),
                pl.BlockSpec((be, 16), lambda i: (i, 0)),
                pl.BlockSpec((be, XW), lambda i: (i, 0)),
                pl.BlockSpec((16, D), lambda i: (0, 0)),
                pl.BlockSpec((1, D), lambda i: (0, 0)),
                pl.BlockSpec((D, D), lambda i: (0, 0)),
                pl.BlockSpec((1, D), lambda i: (0, 0)),
                pl.BlockSpec((1, D), lambda i: (0, 0)),
                pl.BlockSpec((1, 1), lambda i: (0, 0)),
            ],
            out_specs=[
                pl.BlockSpec((be, D), lambda i: (i, 0)),
                pl.BlockSpec((be, XW), lambda i: (i, 0)),
            ],
            out_shape=[
                jax.ShapeDtypeStruct((E2, D), f32),
                jax.ShapeDtypeStruct((E2, XW), f32),
            ],
        )(pre, edge_attr[sl], diff, w1c, b1r, wx, bx,
          px_w2.reshape(1, D), px_b2.reshape(1, 1))

        msum_p, xacc_p = pl.kernel(
            sc2_body,
            compiler_params=sc_params,
            out_type=[
                jax.ShapeDtypeStruct((NC, NPAD, D), f32),
                jax.ShapeDtypeStruct((NC, NPAD, XW), f32),
            ],
            mesh=mesh,
            scratch_types=(set2_types * 2
                           + [pltpu.SemaphoreType.DMA] * 2
                           + [pltpu.VMEM_SHARED((NPAD, D), f32),
                              pltpu.VMEM_SHARED((NPAD, XW), f32)]),
        )(m1, step, u2, v2)
        msum_ps.append(msum_p)
        xacc_ps.append(xacc_p)

    h_new, xn16 = pl.pallas_call(
        _make_node_body(SEG),
        grid=(nb,),
        in_specs=(
            [pl.BlockSpec((bn, D), lambda i: (i, 0)),
             pl.BlockSpec((bn, XW), lambda i: (i, 0))]
            + [pl.BlockSpec((NC, bn, D), lambda i: (0, i, 0))] * SEG
            + [pl.BlockSpec((NC, bn, XW), lambda i: (0, i, 0))] * SEG
            + [pl.BlockSpec((D, D), lambda i: (0, 0)),
               pl.BlockSpec((1, D), lambda i: (0, 0)),
               pl.BlockSpec((D, D), lambda i: (0, 0)),
               pl.BlockSpec((D, D), lambda i: (0, 0)),
               pl.BlockSpec((1, D), lambda i: (0, 0)),
               pl.BlockSpec((D, D), lambda i: (0, 0)),
               pl.BlockSpec((1, D), lambda i: (0, 0)),
               pl.BlockSpec((1, D), lambda i: (0, 0)),
               pl.BlockSpec((1, D), lambda i: (0, 0)),
               pl.BlockSpec((1, D), lambda i: (0, 0)),
               pl.BlockSpec((1, D), lambda i: (0, 0))]
        ),
        out_specs=[
            pl.BlockSpec((bn, D), lambda i: (i, 0)),
            pl.BlockSpec((bn, XW), lambda i: (i, 0)),
        ],
        out_shape=[
            jax.ShapeDtypeStruct((N, D), f32),
            jax.ShapeDtypeStruct((N, XW), f32),
        ],
    )(h, xp, *msum_ps, *xacc_ps,
      pm_w2, pm_b2.reshape(1, D),
      ph_w1[:D], ph_w1[D:], ph_b1.reshape(1, D), ph_w2, ph_b2.reshape(1, D),
      lnh_g.reshape(1, D), lnh_b.reshape(1, D),
      lnm_g.reshape(1, D), lnm_b.reshape(1, D))

    return xn16[:, :3], h_new


# speedup vs baseline: 1.1174x; 1.0273x over previous
import functools

import jax
import jax.numpy as jnp
from jax import lax
from jax.experimental import pallas as pl
from jax.experimental.pallas import tpu as pltpu
from jax.experimental.pallas import tpu_sc as plsc

N = 10000
E = 320000
D = 128
CHUNK = 64
NCHUNKS = E // CHUNK
NC, NS = 2, 16
NW = NC * NS
KMAX = (NCHUNKS + NW - 1) // NW
CH1 = 128
NCH1 = E // CH1
SLOTS1 = 81
NPAD = 10240
ROWS_PER_TILE = NPAD // NS
XW = 16
AW = D + XW


def _ln_rows(xb, g, b, eps=1e-5):
    mu = jnp.mean(xb, axis=-1, keepdims=True)
    var = jnp.mean((xb - mu) ** 2, axis=-1, keepdims=True)
    return (xb - mu) / jnp.sqrt(var + eps) * g + b


def _prep_node_body(h_ref, w1a_ref, w1b_ref, a_ref, b_ref):
    h = h_ref[...]
    a_ref[...] = jnp.dot(h, w1a_ref[...], preferred_element_type=jnp.float32)
    b_ref[...] = jnp.dot(h, w1b_ref[...], preferred_element_type=jnp.float32)


def _fold_w_body(w2_ref, b2_ref, pxw1_ref, pxb1_ref, wx_ref, bx_ref):
    wx_ref[...] = jnp.dot(w2_ref[...], pxw1_ref[...],
                          preferred_element_type=jnp.float32)
    bx_ref[...] = jnp.dot(b2_ref[...], pxw1_ref[...],
                          preferred_element_type=jnp.float32) + pxb1_ref[...]


def _make_sc1_body(nch1, nbody):
  def _sc1_body(a_h, b_h, xp_h, u_h, v_h, pre_h, df_h, *scr):
    sets = (scr[0:7], scr[7:14], scr[14:21])
    sg = scr[21:24]
    sw = scr[24:27]
    wid = lax.axis_index("s") * NC + lax.axis_index("c")

    def fire_gathers(k, b):
        iu, iv, au, bv, xu, xv, df = sets[b]
        chunk = k * NW + wid

        @pl.when(chunk < nch1)
        def _():
            pltpu.sync_copy(u_h.at[chunk], iu)
            pltpu.sync_copy(v_h.at[chunk], iv)
            pltpu.async_copy(a_h.at[iu], au, sg[b])
            pltpu.async_copy(b_h.at[iv], bv, sg[b])
            pltpu.async_copy(xp_h.at[iu], xu, sg[b])
            pltpu.async_copy(xp_h.at[iv], xv, sg[b])

    def drain_writes(k, b):
        iu, iv, au, bv, xu, xv, df = sets[b]
        chunk = jnp.maximum(k, 0) * NW + wid

        @pl.when((k >= 0) & (chunk < nch1))
        def _():
            row = pl.ds(chunk * CH1, CH1)
            pltpu.make_async_copy(au, pre_h.at[row], sw[b]).wait()
            pltpu.make_async_copy(df, df_h.at[row], sw[b]).wait()

    def process(k, b):
        iu, iv, au, bv, xu, xv, df = sets[b]
        chunk = k * NW + wid

        @pl.when(chunk < nch1)
        def _():
            pltpu.make_async_copy(a_h.at[iu], au, sg[b]).wait()
            pltpu.make_async_copy(b_h.at[iv], bv, sg[b]).wait()
            pltpu.make_async_copy(xp_h.at[iu], xu, sg[b]).wait()
            pltpu.make_async_copy(xp_h.at[iv], xv, sg[b]).wait()

            def _row(r, _):
                for j in range(D // 16):
                    sl = pl.ds(j * 16, 16)
                    au[r, sl] = au[r, sl] + bv[r, sl]
                df[r, :] = xu[r, :] - xv[r, :]
                return 0
            lax.fori_loop(0, CH1, _row, 0)

            row = pl.ds(chunk * CH1, CH1)
            pltpu.async_copy(au, pre_h.at[row], sw[b])
            pltpu.async_copy(df, df_h.at[row], sw[b])

    fire_gathers(0, 0)
    fire_gathers(1, 1)

    def _body(t, _):
        for b in range(3):
            k = t * 3 + b
            process(k, b)
            drain_writes(k - 1, (b + 2) % 3)
            fire_gathers(k + 2, (b + 2) % 3)
        return 0
    lax.fori_loop(0, nbody, _body, 0)

  return _sc1_body


def _edge_x_body(pre_ref, ea_ref, df_ref, w1c_ref, b1_ref, wx_ref, bx_ref,
                 w2r_ref, b2_ref, m1_ref, stp_ref):
    pre = (pre_ref[...]
           + jnp.dot(ea_ref[...], w1c_ref[...],
                     preferred_element_type=jnp.float32)
           + b1_ref[...])
    m1 = pre * jax.nn.sigmoid(pre)
    m1_ref[...] = m1
    t = jnp.dot(m1, wx_ref[...], preferred_element_type=jnp.float32) + bx_ref[...]
    t = t * jax.nn.sigmoid(t)
    w = jnp.tanh(jnp.sum(t * w2r_ref[...], axis=-1, keepdims=True)
                 + b2_ref[...]) * 0.1
    df = df_ref[...]
    dist = jnp.maximum(jnp.sqrt(jnp.sum(df * df, axis=-1, keepdims=True)), 1e-8)
    lane = lax.broadcasted_iota(jnp.int32, df.shape, 1)
    ind = jnp.where((lane == 3) | (lane == 4), 1.0, 0.0)
    stp_ref[...] = w * df / dist + ind


def _make_sc2_body(nchunks, nbody2):
  def _sc2_body(m1_h, st_h, u_h, v_h, ms_h, xa_h, *scr):
    sets = (scr[0:5], scr[5:10])
    sl_sem = scr[10:12]
    ms_s, xa_s = scr[12], scr[13]
    cid = lax.axis_index("c")
    sid = lax.axis_index("s")
    wid = sid * NC + cid

    zero16 = jnp.zeros((16,), jnp.float32)
    m1c0, st0 = sets[0][2], sets[0][3]

    def _zrow(r, _):
        for j in range(D // 16):
            m1c0[r, pl.ds(j * 16, 16)] = zero16
        st0[r, :] = zero16
        return 0
    lax.fori_loop(0, CHUNK, _zrow, 0)
    for i in range(ROWS_PER_TILE // CHUNK):
        dst = pl.ds(sid * ROWS_PER_TILE + i * CHUNK, CHUNK)
        pltpu.sync_copy(m1c0.at[pl.ds(0, CHUNK), pl.ds(0, D)], ms_s.at[dst])
        pltpu.sync_copy(st0.at[pl.ds(0, CHUNK), pl.ds(0, XW)], xa_s.at[dst])

    plsc.subcore_barrier()

    lane = lax.broadcasted_iota(jnp.int32, (16,), 0)
    negmask = jnp.where(lane < 3, -1.0,
                        jnp.where(lane == 3, 1.0, 0.0)).astype(jnp.float32)

    def fire_loads(k, b):
        iu, iv, m1c, st16, stn16 = sets[b]
        chunk = k * NW + wid

        @pl.when(chunk < nchunks)
        def _():
            pltpu.sync_copy(u_h.at[chunk], iu)
            pltpu.sync_copy(v_h.at[chunk], iv)
            row = pl.ds(chunk * CHUNK, CHUNK)
            pltpu.async_copy(m1_h.at[row], m1c, sl_sem[b])
            pltpu.async_copy(st_h.at[row], st16, sl_sem[b])

    def work(k, b):
        iu, iv, m1c, st16, stn16 = sets[b]
        chunk = k * NW + wid

        @pl.when(chunk < nchunks)
        def _():
            row = pl.ds(chunk * CHUNK, CHUNK)
            pltpu.make_async_copy(m1_h.at[row], m1c, sl_sem[b]).wait()
            pltpu.make_async_copy(st_h.at[row], st16, sl_sem[b]).wait()

            def _row(r, _):
                stn16[r, :] = st16[r, :] * negmask
                return 0
            lax.fori_loop(0, CHUNK, _row, 0)

            pltpu.sync_copy(m1c, ms_s.at[iu], add=True)
            pltpu.sync_copy(st16, xa_s.at[iu], add=True)
            pltpu.sync_copy(stn16, xa_s.at[iv], add=True)

    fire_loads(0, 0)

    def _body(t, _):
        for b in range(2):
            k = t * 2 + b
            fire_loads(k + 1, 1 - b)
            work(k, b)
        return 0
    lax.fori_loop(0, nbody2, _body, 0)

    plsc.subcore_barrier()
    src = pl.ds(sid * ROWS_PER_TILE, ROWS_PER_TILE)
    pltpu.sync_copy(ms_s.at[src], ms_h.at[cid, src])
    pltpu.sync_copy(xa_s.at[src], xa_h.at[cid, src])

  return _sc2_body


def _make_node_body(seg):
  def _node_body(h_ref, xp_ref, *rest):
    ms_refs = rest[0:seg]
    xa_refs = rest[seg:2 * seg]
    (w2_ref, b2_ref, wha_ref, whb_ref, bh1_ref, wh2_ref, bh2_ref,
     lnhg_ref, lnhb_ref, lnmg_ref, lnmb_ref, hn_ref, xn_ref) = rest[2 * seg:]
    ms = ms_refs[0][0] + ms_refs[0][1]
    xa = xa_refs[0][0] + xa_refs[0][1]
    for i in range(1, seg):
        ms = ms + ms_refs[i][0] + ms_refs[i][1]
        xa = xa + xa_refs[i][0] + xa_refs[i][1]
    deg = xa[:, 4:5]
    m_sum = (jnp.dot(ms, w2_ref[...], preferred_element_type=jnp.float32)
             + deg * b2_ref[...])
    m_i = m_sum / jnp.maximum(deg, 1.0)
    m_i = _ln_rows(m_i, lnmg_ref[...], lnmb_ref[...])
    h = h_ref[...]
    h_in = _ln_rows(h, lnhg_ref[...], lnhb_ref[...])
    pre = (jnp.dot(h_in, wha_ref[...], preferred_element_type=jnp.float32)
           + jnp.dot(m_i, whb_ref[...], preferred_element_type=jnp.float32)
           + bh1_ref[...])
    upd = (jnp.dot(pre * jax.nn.sigmoid(pre), wh2_ref[...],
                   preferred_element_type=jnp.float32) + bh2_ref[...])
    hn_ref[...] = h + 0.3 * upd

    degx = jnp.maximum(xa[:, 3:4], 1.0)
    xn_ref[...] = xp_ref[...] + xa / degx

  return _node_body


def kernel(x, h, edge_index, edge_attr, pm_w1, pm_b1, pm_w2, pm_b2,
           ph_w1, ph_b1, ph_w2, ph_b2, px_w1, px_b1, px_w2, px_b2,
           lnh_g, lnh_b, lnm_g, lnm_b):
    f32 = jnp.float32
    xp = jnp.pad(x, ((0, 0), (0, XW - 3)))

    w1a = pm_w1[:D]
    w1b = pm_w1[D:2 * D]
    w1c = pm_w1[2 * D:]
    b1r = pm_b1.reshape(1, D)

    nb = 10
    bn = N // nb
    a_nd, b_nd = pl.pallas_call(
        _prep_node_body,
        grid=(nb,),
        in_specs=[
            pl.BlockSpec((bn, D), lambda i: (i, 0)),
            pl.BlockSpec((D, D), lambda i: (0, 0)),
            pl.BlockSpec((D, D), lambda i: (0, 0)),
        ],
        out_specs=[
            pl.BlockSpec((bn, D), lambda i: (i, 0)),
            pl.BlockSpec((bn, D), lambda i: (i, 0)),
        ],
        out_shape=[
            jax.ShapeDtypeStruct((N, D), f32),
            jax.ShapeDtypeStruct((N, D), f32),
        ],
    )(h, w1a, w1b)

    eb = 200
    be = E // eb

    wx, bx = pl.pallas_call(
        _fold_w_body,
        out_shape=[
            jax.ShapeDtypeStruct((D, D), f32),
            jax.ShapeDtypeStruct((1, D), f32),
        ],
    )(pm_w2, pm_b2.reshape(1, D), px_w1, px_b1.reshape(1, D))

    mesh = plsc.VectorSubcoreMesh(core_axis_name="c", subcore_axis_name="s",
                                  num_cores=NC, num_subcores=NS)
    sc_params = pltpu.CompilerParams(use_tc_tiling_on_sc=False)
    SEG = 4
    E2 = E // SEG
    nch1_h = E2 // CH1
    kmax1_h = (nch1_h + NW - 1) // NW
    nbody1_h = (kmax1_h + 2 + 2) // 3
    nch2_h = E2 // CHUNK
    kmax2_h = (nch2_h + NW - 1) // NW
    u_all = edge_index[0].astype(jnp.int32)
    v_all = edge_index[1].astype(jnp.int32)
    set_types = [
        pltpu.VMEM((CH1,), jnp.int32),
        pltpu.VMEM((CH1,), jnp.int32),
        pltpu.VMEM((CH1, D), f32),
        pltpu.VMEM((CH1, D), f32),
        pltpu.VMEM((CH1, XW), f32),
        pltpu.VMEM((CH1, XW), f32),
        pltpu.VMEM((CH1, XW), f32),
    ]
    sc1_body = _make_sc1_body(nch1_h, nbody1_h)
    sc2_body = _make_sc2_body(nch2_h, (kmax2_h + 1) // 2)
    set2_types = [
        pltpu.VMEM((CHUNK,), jnp.int32),
        pltpu.VMEM((CHUNK,), jnp.int32),
        pltpu.VMEM((CHUNK, D), f32),
        pltpu.VMEM((CHUNK, XW), f32),
        pltpu.VMEM((CHUNK, XW), f32),
    ]
    eb = 200 // SEG
    be = E2 // eb

    msum_ps, xacc_ps = [], []
    for hf in range(SEG):
        sl = slice(hf * E2, (hf + 1) * E2)
        u1 = u_all[sl].reshape(nch1_h, CH1)
        v1 = v_all[sl].reshape(nch1_h, CH1)
        u2 = u_all[sl].reshape(nch2_h, CHUNK)
        v2 = v_all[sl].reshape(nch2_h, CHUNK)

        pre, diff = pl.kernel(
            sc1_body,
            compiler_params=sc_params,
            out_type=[
                jax.ShapeDtypeStruct((E2, D), f32),
                jax.ShapeDtypeStruct((E2, XW), f32),
            ],
            mesh=mesh,
            scratch_types=(set_types * 3
                           + [pltpu.SemaphoreType.DMA] * 6),
        )(a_nd, b_nd, xp, u1, v1)

        m1, step = pl.pallas_call(
            _edge_x_body,
            grid=(eb,),
            in_specs=[
                pl.BlockSpec((be, D), lambda i: (i, 0)),
                pl.BlockSpec((be, 16), lambda i: (i, 0)),
                pl.BlockSpec((be, XW), lambda i: (i, 0)),
                pl.BlockSpec((16, D), lambda i: (0, 0)),
                pl.BlockSpec((1, D), lambda i: (0, 0)),
                pl.BlockSpec((D, D), lambda i: (0, 0)),
                pl.BlockSpec((1, D), lambda i: (0, 0)),
                pl.BlockSpec((1, D), lambda i: (0, 0)),
                pl.BlockSpec((1, 1), lambda i: (0, 0)),
            ],
            out_specs=[
                pl.BlockSpec((be, D), lambda i: (i, 0)),
                pl.BlockSpec((be, XW), lambda i: (i, 0)),
            ],
            out_shape=[
                jax.ShapeDtypeStruct((E2, D), f32),
                jax.ShapeDtypeStruct((E2, XW), f32),
            ],
        )(pre, edge_attr[sl], diff, w1c, b1r, wx, bx,
          px_w2.reshape(1, D), px_b2.reshape(1, 1))

        msum_p, xacc_p = pl.kernel(
            sc2_body,
            compiler_params=sc_params,
            out_type=[
                jax.ShapeDtypeStruct((NC, NPAD, D), f32),
                jax.ShapeDtypeStruct((NC, NPAD, XW), f32),
            ],
            mesh=mesh,
            scratch_types=(set2_types * 2
                           + [pltpu.SemaphoreType.DMA] * 2
                           + [pltpu.VMEM_SHARED((NPAD, D), f32),
                              pltpu.VMEM_SHARED((NPAD, XW), f32)]),
        )(m1, step, u2, v2)
        msum_ps.append(msum_p)
        xacc_ps.append(xacc_p)

    h_new, xn16 = pl.pallas_call(
        _make_node_body(SEG),
        grid=(nb,),
        in_specs=(
            [pl.BlockSpec((bn, D), lambda i: (i, 0)),
             pl.BlockSpec((bn, XW), lambda i: (i, 0))]
            + [pl.BlockSpec((NC, bn, D), lambda i: (0, i, 0))] * SEG
            + [pl.BlockSpec((NC, bn, XW), lambda i: (0, i, 0))] * SEG
            + [pl.BlockSpec((D, D), lambda i: (0, 0)),
               pl.BlockSpec((1, D), lambda i: (0, 0)),
               pl.BlockSpec((D, D), lambda i: (0, 0)),
               pl.BlockSpec((D, D), lambda i: (0, 0)),
               pl.BlockSpec((1, D), lambda i: (0, 0)),
               pl.BlockSpec((D, D), lambda i: (0, 0)),
               pl.BlockSpec((1, D), lambda i: (0, 0)),
               pl.BlockSpec((1, D), lambda i: (0, 0)),
               pl.BlockSpec((1, D), lambda i: (0, 0)),
               pl.BlockSpec((1, D), lambda i: (0, 0)),
               pl.BlockSpec((1, D), lambda i: (0, 0))]
        ),
        out_specs=[
            pl.BlockSpec((bn, D), lambda i: (i, 0)),
            pl.BlockSpec((bn, XW), lambda i: (i, 0)),
        ],
        out_shape=[
            jax.ShapeDtypeStruct((N, D), f32),
            jax.ShapeDtypeStruct((N, XW), f32),
        ],
    )(h, xp, *msum_ps, *xacc_ps,
      pm_w2, pm_b2.reshape(1, D),
      ph_w1[:D], ph_w1[D:], ph_b1.reshape(1, D), ph_w2, ph_b2.reshape(1, D),
      lnh_g.reshape(1, D), lnh_b.reshape(1, D),
      lnm_g.reshape(1, D), lnm_b.reshape(1, D))

    return xn16[:, :3], h_new
